# Initial kernel scaffold; baseline (speedup 1.0000x reference)
import jax
import jax.numpy as jnp
from jax.experimental import pallas as pl


def kernel(h, adj, weight, attn_l_w, attn_r_w, b):
    raise NotImplementedError("write your pallas kernel here")


# speedup vs baseline: 1.5346x; 1.5346x over previous
import functools

import jax
import jax.numpy as jnp
from jax.experimental import pallas as pl

_N, _DIN, _DOUT = 4096, 128, 64
_ROW_TILE = 512


def _proj_kernel(h_ref, w_ref, al_ref, ar_ref, x_ref, el_ref, ert_ref):
    x = jnp.dot(h_ref[:], w_ref[:], preferred_element_type=jnp.float32)
    x_ref[:] = x
    el_ref[:] = jnp.sum(x * al_ref[:], axis=1, keepdims=True)
    ert_ref[:] = jax.lax.dot_general(
        ar_ref[:], x, (((1,), (1,)), ((), ())),
        preferred_element_type=jnp.float32)


def _attn_kernel(el_ref, ert_ref, adj_ref, x_ref, b_ref, out_ref):
    z = el_ref[:] + ert_ref[:]
    z = jnp.where(z >= 0, z, 0.2 * z)
    a = jnp.where(adj_ref[:] != 0, jnp.exp(z), 0.0)
    s = jnp.sum(a, axis=1, keepdims=True)
    o = jnp.dot(a, x_ref[:], preferred_element_type=jnp.float32)
    out_ref[:] = o / jnp.maximum(s, 1e-12) + b_ref[:]


@functools.partial(jax.jit, static_argnames=())
def kernel(h, adj, weight, attn_l_w, attn_r_w, b):
    n, din = h.shape
    dout = weight.shape[1]

    x, el, ert = pl.pallas_call(
        _proj_kernel,
        out_shape=(
            jax.ShapeDtypeStruct((n, dout), jnp.float32),
            jax.ShapeDtypeStruct((n, 1), jnp.float32),
            jax.ShapeDtypeStruct((1, n), jnp.float32),
        ),
    )(h, weight, attn_l_w, attn_r_w)

    r = _ROW_TILE
    out = pl.pallas_call(
        _attn_kernel,
        grid=(n // r,),
        in_specs=[
            pl.BlockSpec((r, 1), lambda i: (i, 0)),
            pl.BlockSpec((1, n), lambda i: (0, 0)),
            pl.BlockSpec((r, n), lambda i: (i, 0)),
            pl.BlockSpec((n, dout), lambda i: (0, 0)),
            pl.BlockSpec((1, dout), lambda i: (0, 0)),
        ],
        out_specs=pl.BlockSpec((r, dout), lambda i: (i, 0)),
        out_shape=jax.ShapeDtypeStruct((n, dout), jnp.float32),
    )(el, ert, adj, x, b.reshape(1, dout))
    return out


# speedup vs baseline: 1.6988x; 1.1070x over previous
import functools

import jax
import jax.numpy as jnp
from jax.experimental import pallas as pl

_ROW_TILE = 512


def _proj_kernel(h_ref, w_ref, al_ref, ar_ref, x_ref, p_ref, q_ref,
                 ut_ref, vt_ref):
    x = jnp.dot(h_ref[:], w_ref[:], preferred_element_type=jnp.float32)
    x_ref[:] = x
    el = jnp.sum(x * al_ref[:], axis=1, keepdims=True)
    p_ref[:] = jnp.exp(el)
    q_ref[:] = jnp.exp(0.2 * el)
    ert = jax.lax.dot_general(
        ar_ref[:], x, (((1,), (1,)), ((), ())),
        preferred_element_type=jnp.float32)
    ut_ref[:] = jnp.exp(ert)
    vt_ref[:] = jnp.exp(0.2 * ert)


def _attn_kernel(p_ref, q_ref, ut_ref, vt_ref, adj_ref, x_ref, b_ref,
                 out_ref):
    a = jnp.maximum(p_ref[:] * ut_ref[:], q_ref[:] * vt_ref[:]) * adj_ref[:]
    s = jnp.sum(a, axis=1, keepdims=True)
    o = jnp.dot(a, x_ref[:], preferred_element_type=jnp.float32)
    out_ref[:] = o / jnp.maximum(s, 1e-12) + b_ref[:]


@functools.partial(jax.jit, static_argnames=())
def kernel(h, adj, weight, attn_l_w, attn_r_w, b):
    n, din = h.shape
    dout = weight.shape[1]

    x, p, q, ut, vt = pl.pallas_call(
        _proj_kernel,
        out_shape=(
            jax.ShapeDtypeStruct((n, dout), jnp.float32),
            jax.ShapeDtypeStruct((n, 1), jnp.float32),
            jax.ShapeDtypeStruct((n, 1), jnp.float32),
            jax.ShapeDtypeStruct((1, n), jnp.float32),
            jax.ShapeDtypeStruct((1, n), jnp.float32),
        ),
    )(h, weight, attn_l_w, attn_r_w)

    r = _ROW_TILE
    out = pl.pallas_call(
        _attn_kernel,
        grid=(n // r,),
        in_specs=[
            pl.BlockSpec((r, 1), lambda i: (i, 0)),
            pl.BlockSpec((r, 1), lambda i: (i, 0)),
            pl.BlockSpec((1, n), lambda i: (0, 0)),
            pl.BlockSpec((1, n), lambda i: (0, 0)),
            pl.BlockSpec((r, n), lambda i: (i, 0)),
            pl.BlockSpec((n, dout), lambda i: (0, 0)),
            pl.BlockSpec((1, dout), lambda i: (0, 0)),
        ],
        out_specs=pl.BlockSpec((r, dout), lambda i: (i, 0)),
        out_shape=jax.ShapeDtypeStruct((n, dout), jnp.float32),
    )(p, q, ut, vt, adj, x, b.reshape(1, dout))
    return out
